# baseline (speedup 1.0000x reference)
import functools

import jax
import jax.numpy as jnp
from jax import lax
from jax.experimental import pallas as pl
from jax.experimental.pallas import tpu as pltpu
from jax.experimental.pallas import tpu_sc as plsc

N_ASSETS = 64
N_REGIMES = 8
HIDDEN = 256

_SC_CORES = 2
_SC_SUBCORES = 16
_SC_WORKERS = _SC_CORES * _SC_SUBCORES
_FILL_ROWS = 128


def _tc_body(x_ref, w1_ref, b1_ref, w2_ref, b2_ref, wh_ref, bh_ref,
             mean_ref, *, tile_rows):
    xt = x_ref[...]
    reg = xt[:, HIDDEN - 1:HIDDEN].astype(jnp.int32)
    h = jnp.dot(xt, w1_ref[...], preferred_element_type=jnp.float32)
    h = jnp.maximum(h + b1_ref[...], 0.0)
    f = jnp.dot(h, w2_ref[...], preferred_element_type=jnp.float32)
    f = jnp.maximum(f + b2_ref[...], 0.0)
    oa = jnp.dot(f, wh_ref[...], preferred_element_type=jnp.float32)
    oa = oa + bh_ref[...]
    wide = N_REGIMES * N_ASSETS
    col = jax.lax.broadcasted_iota(jnp.int32, (tile_rows, wide), 1)
    sel = jnp.where(col // N_ASSETS == reg, oa, 0.0)
    fold_c = jax.lax.broadcasted_iota(jnp.int32, (wide, N_ASSETS), 0)
    fold_a = jax.lax.broadcasted_iota(jnp.int32, (wide, N_ASSETS), 1)
    fold = (fold_c % N_ASSETS == fold_a).astype(jnp.float32)
    mean = jnp.dot(sel, fold, preferred_element_type=jnp.float32)
    mean_ref[...] = mean * 0.1


def _sc_std_body(ls_hbm, std_hbm, ls_v, buf_v, *, rows_per_worker):
    wid = lax.axis_index("s") * _SC_CORES + lax.axis_index("c")
    pltpu.sync_copy(ls_hbm, ls_v)
    svals = []
    for k in range(N_ASSETS // 16):
        v = ls_v[pl.ds(16 * k, 16)]
        svals.append(jnp.minimum(jnp.maximum(jnp.exp(v), 1e-3), 1.0))

    def fill_row(r, carry):
        for k in range(N_ASSETS // 16):
            buf_v[r, pl.ds(16 * k, 16)] = svals[k]
        return carry

    lax.fori_loop(0, _FILL_ROWS, fill_row, 0)
    base = wid * rows_per_worker
    for m in range(rows_per_worker // _FILL_ROWS):
        pltpu.sync_copy(buf_v, std_hbm.at[pl.ds(base + m * _FILL_ROWS,
                                                _FILL_ROWS)])


def kernel(x, W1, b1, W2, b2, Wh, bh, log_std):
    batch, in_dim = x.shape
    tile_rows = 2048
    grid = (batch // tile_rows,)

    wh_all = jnp.transpose(Wh, (1, 0, 2)).reshape(HIDDEN, N_REGIMES * N_ASSETS)
    bh_all = bh.reshape(1, N_REGIMES * N_ASSETS)
    b1r = b1.reshape(1, HIDDEN)
    b2r = b2.reshape(1, HIDDEN)

    const = lambda *_: (0, 0)
    mean, = pl.pallas_call(
        functools.partial(_tc_body, tile_rows=tile_rows),
        grid=grid,
        in_specs=[
            pl.BlockSpec((tile_rows, in_dim), lambda i: (i, 0)),
            pl.BlockSpec((in_dim, HIDDEN), const),
            pl.BlockSpec((1, HIDDEN), const),
            pl.BlockSpec((HIDDEN, HIDDEN), const),
            pl.BlockSpec((1, HIDDEN), const),
            pl.BlockSpec((HIDDEN, N_REGIMES * N_ASSETS), const),
            pl.BlockSpec((1, N_REGIMES * N_ASSETS), const),
        ],
        out_specs=[
            pl.BlockSpec((tile_rows, N_ASSETS), lambda i: (i, 0)),
        ],
        out_shape=[
            jax.ShapeDtypeStruct((batch, N_ASSETS), jnp.float32),
        ],
        compiler_params=pltpu.CompilerParams(
            dimension_semantics=("arbitrary",),
        ),
    )(x, W1, b1r, W2, b2r, wh_all, bh_all)

    rows_per_worker = batch // _SC_WORKERS
    std_kernel = pl.kernel(
        functools.partial(_sc_std_body, rows_per_worker=rows_per_worker),
        mesh=plsc.VectorSubcoreMesh(core_axis_name="c", subcore_axis_name="s"),
        out_type=jax.ShapeDtypeStruct((batch, N_ASSETS), jnp.float32),
        scratch_types=[
            pltpu.VMEM((N_ASSETS,), jnp.float32),
            pltpu.VMEM((_FILL_ROWS, N_ASSETS), jnp.float32),
        ],
    )
    std = std_kernel(log_std)
    return (mean, std)


# speedup vs baseline: 1.4493x; 1.4493x over previous
import functools

import jax
import jax.numpy as jnp
from jax.experimental import pallas as pl
from jax.experimental.pallas import tpu as pltpu

N_ASSETS = 64
N_REGIMES = 8
HIDDEN = 256


def _chain(xt, w1, b1, w2, b2, wh, bh, rows):
    reg = xt[:, HIDDEN - 1:HIDDEN].astype(jnp.int32)
    h = jnp.dot(xt, w1, preferred_element_type=jnp.float32)
    h = jnp.maximum(h + b1, 0.0)
    f = jnp.dot(h, w2, preferred_element_type=jnp.float32)
    f = jnp.maximum(f + b2, 0.0)
    oa = jnp.dot(f, wh, preferred_element_type=jnp.float32)
    oa = oa + bh
    wide = N_REGIMES * N_ASSETS
    col = jax.lax.broadcasted_iota(jnp.int32, (rows, wide), 1)
    sel = jnp.where(col // N_ASSETS == reg, oa, 0.0)
    fold_c = jax.lax.broadcasted_iota(jnp.int32, (wide, N_ASSETS), 0)
    fold_a = jax.lax.broadcasted_iota(jnp.int32, (wide, N_ASSETS), 1)
    fold = jnp.where(fold_c % N_ASSETS == fold_a, 0.1, 0.0)
    return jnp.dot(sel, fold, preferred_element_type=jnp.float32)


def _body(x_ref, w1_ref, b1_ref, w2_ref, b2_ref, wh_ref, bh_ref, ls_ref,
          mean_ref, std_ref, *, tile_rows):
    half = tile_rows // 2
    args = (w1_ref[...], b1_ref[...], w2_ref[...], b2_ref[...],
            wh_ref[...], bh_ref[...])
    mean_ref[:half, :] = _chain(x_ref[:half, :], *args, half)
    mean_ref[half:, :] = _chain(x_ref[half:, :], *args, half)
    std = jnp.clip(jnp.exp(ls_ref[...]), 1e-3, 1.0)
    std_ref[...] = jnp.broadcast_to(std, (tile_rows, N_ASSETS))


def kernel(x, W1, b1, W2, b2, Wh, bh, log_std):
    batch, in_dim = x.shape
    tile_rows = 2048
    grid = (batch // tile_rows,)

    wh_all = jnp.transpose(Wh, (1, 0, 2)).reshape(HIDDEN, N_REGIMES * N_ASSETS)
    bh_all = bh.reshape(1, N_REGIMES * N_ASSETS)
    b1r = b1.reshape(1, HIDDEN)
    b2r = b2.reshape(1, HIDDEN)
    lsr = log_std.reshape(1, N_ASSETS)

    const = lambda *_: (0, 0)
    mean, std = pl.pallas_call(
        functools.partial(_body, tile_rows=tile_rows),
        grid=grid,
        in_specs=[
            pl.BlockSpec((tile_rows, in_dim), lambda i: (i, 0)),
            pl.BlockSpec((in_dim, HIDDEN), const),
            pl.BlockSpec((1, HIDDEN), const),
            pl.BlockSpec((HIDDEN, HIDDEN), const),
            pl.BlockSpec((1, HIDDEN), const),
            pl.BlockSpec((HIDDEN, N_REGIMES * N_ASSETS), const),
            pl.BlockSpec((1, N_REGIMES * N_ASSETS), const),
            pl.BlockSpec((1, N_ASSETS), const),
        ],
        out_specs=[
            pl.BlockSpec((tile_rows, N_ASSETS), lambda i: (i, 0)),
            pl.BlockSpec((tile_rows, N_ASSETS), lambda i: (i, 0)),
        ],
        out_shape=[
            jax.ShapeDtypeStruct((batch, N_ASSETS), jnp.float32),
            jax.ShapeDtypeStruct((batch, N_ASSETS), jnp.float32),
        ],
        compiler_params=pltpu.CompilerParams(
            dimension_semantics=("arbitrary",),
        ),
    )(x, W1, b1r, W2, b2r, wh_all, bh_all, lsr)
    return (mean, std)


# speedup vs baseline: 1.4795x; 1.0209x over previous
import functools

import jax
import jax.numpy as jnp
from jax.experimental import pallas as pl
from jax.experimental.pallas import tpu as pltpu

N_ASSETS = 64
N_REGIMES = 8
HIDDEN = 256


def _chain(xt, w1, b1, w2, b2, wh, bh, rows):
    reg = xt[:, HIDDEN - 1:HIDDEN].astype(jnp.int32)
    h = jnp.dot(xt, w1, preferred_element_type=jnp.float32)
    h = jnp.maximum(h + b1, 0.0)
    f = jnp.dot(h, w2, preferred_element_type=jnp.float32)
    f = jnp.maximum(f + b2, 0.0)
    oa = jnp.dot(f, wh, preferred_element_type=jnp.float32)
    oa = oa + bh
    wide = N_REGIMES * N_ASSETS
    col = jax.lax.broadcasted_iota(jnp.int32, (rows, wide), 1)
    sel = jnp.where(col // N_ASSETS == reg, oa, 0.0)
    fold_c = jax.lax.broadcasted_iota(jnp.int32, (wide, N_ASSETS), 0)
    fold_a = jax.lax.broadcasted_iota(jnp.int32, (wide, N_ASSETS), 1)
    fold = jnp.where(fold_c % N_ASSETS == fold_a, 0.1, 0.0)
    return jnp.dot(sel, fold, preferred_element_type=jnp.float32)


def _body(x_ref, w1_ref, b1_ref, w2_ref, b2_ref, wh_ref, bh_ref, ls_ref,
          mean_ref, std_ref, *, tile_rows):
    quarter = tile_rows // 4
    args = (w1_ref[...], b1_ref[...], w2_ref[...], b2_ref[...],
            wh_ref[...], bh_ref[...])
    for q in range(4):
        sl = slice(q * quarter, (q + 1) * quarter)
        mean_ref[sl, :] = _chain(x_ref[sl, :], *args, quarter)
    std = jnp.clip(jnp.exp(ls_ref[...]), 1e-3, 1.0)
    std_ref[...] = jnp.broadcast_to(std, (tile_rows, N_ASSETS))


def kernel(x, W1, b1, W2, b2, Wh, bh, log_std):
    batch, in_dim = x.shape
    tile_rows = 2048
    grid = (batch // tile_rows,)

    wh_all = jnp.transpose(Wh, (1, 0, 2)).reshape(HIDDEN, N_REGIMES * N_ASSETS)
    bh_all = bh.reshape(1, N_REGIMES * N_ASSETS)
    b1r = b1.reshape(1, HIDDEN)
    b2r = b2.reshape(1, HIDDEN)
    lsr = log_std.reshape(1, N_ASSETS)

    const = lambda *_: (0, 0)
    mean, std = pl.pallas_call(
        functools.partial(_body, tile_rows=tile_rows),
        grid=grid,
        in_specs=[
            pl.BlockSpec((tile_rows, in_dim), lambda i: (i, 0)),
            pl.BlockSpec((in_dim, HIDDEN), const),
            pl.BlockSpec((1, HIDDEN), const),
            pl.BlockSpec((HIDDEN, HIDDEN), const),
            pl.BlockSpec((1, HIDDEN), const),
            pl.BlockSpec((HIDDEN, N_REGIMES * N_ASSETS), const),
            pl.BlockSpec((1, N_REGIMES * N_ASSETS), const),
            pl.BlockSpec((1, N_ASSETS), const),
        ],
        out_specs=[
            pl.BlockSpec((tile_rows, N_ASSETS), lambda i: (i, 0)),
            pl.BlockSpec((tile_rows, N_ASSETS), lambda i: (i, 0)),
        ],
        out_shape=[
            jax.ShapeDtypeStruct((batch, N_ASSETS), jnp.float32),
            jax.ShapeDtypeStruct((batch, N_ASSETS), jnp.float32),
        ],
        compiler_params=pltpu.CompilerParams(
            dimension_semantics=("arbitrary",),
        ),
    )(x, W1, b1r, W2, b2r, wh_all, bh_all, lsr)
    return (mean, std)


# speedup vs baseline: 1.5056x; 1.0176x over previous
import functools

import jax
import jax.numpy as jnp
from jax.experimental import pallas as pl
from jax.experimental.pallas import tpu as pltpu

N_ASSETS = 64
N_REGIMES = 8
HIDDEN = 256


def _chain(xt, w1, b1, w2, b2, wh, bh, rows):
    reg = xt[:, HIDDEN - 1:HIDDEN].astype(jnp.int32)
    h = jnp.dot(xt, w1, preferred_element_type=jnp.float32)
    h = jnp.maximum(h + b1, 0.0)
    f = jnp.dot(h, w2, preferred_element_type=jnp.float32)
    f = jnp.maximum(f + b2, 0.0)
    oa = jnp.dot(f, wh, preferred_element_type=jnp.float32)
    oa = oa + bh
    wide = N_REGIMES * N_ASSETS
    col = jax.lax.broadcasted_iota(jnp.int32, (rows, wide), 1)
    sel = jnp.where(col // N_ASSETS == reg, oa, 0.0)
    fold_c = jax.lax.broadcasted_iota(jnp.int32, (wide, N_ASSETS), 0)
    fold_a = jax.lax.broadcasted_iota(jnp.int32, (wide, N_ASSETS), 1)
    fold = jnp.where(fold_c % N_ASSETS == fold_a, 0.1, 0.0)
    return jnp.dot(sel, fold, preferred_element_type=jnp.float32)


def _body(x_ref, w1_ref, b1_ref, w2_ref, b2_ref, wh_ref, bh_ref, ls_ref,
          mean_ref, std_ref, *, tile_rows):
    quarter = tile_rows // 8
    args = (w1_ref[...], b1_ref[...], w2_ref[...], b2_ref[...],
            wh_ref[...], bh_ref[...])
    for q in range(8):
        sl = slice(q * quarter, (q + 1) * quarter)
        mean_ref[sl, :] = _chain(x_ref[sl, :], *args, quarter)
    std = jnp.clip(jnp.exp(ls_ref[...]), 1e-3, 1.0)
    std_ref[...] = jnp.broadcast_to(std, (tile_rows, N_ASSETS))


def kernel(x, W1, b1, W2, b2, Wh, bh, log_std):
    batch, in_dim = x.shape
    tile_rows = 4096
    grid = (batch // tile_rows,)

    wh_all = jnp.transpose(Wh, (1, 0, 2)).reshape(HIDDEN, N_REGIMES * N_ASSETS)
    bh_all = bh.reshape(1, N_REGIMES * N_ASSETS)
    b1r = b1.reshape(1, HIDDEN)
    b2r = b2.reshape(1, HIDDEN)
    lsr = log_std.reshape(1, N_ASSETS)

    const = lambda *_: (0, 0)
    mean, std = pl.pallas_call(
        functools.partial(_body, tile_rows=tile_rows),
        grid=grid,
        in_specs=[
            pl.BlockSpec((tile_rows, in_dim), lambda i: (i, 0)),
            pl.BlockSpec((in_dim, HIDDEN), const),
            pl.BlockSpec((1, HIDDEN), const),
            pl.BlockSpec((HIDDEN, HIDDEN), const),
            pl.BlockSpec((1, HIDDEN), const),
            pl.BlockSpec((HIDDEN, N_REGIMES * N_ASSETS), const),
            pl.BlockSpec((1, N_REGIMES * N_ASSETS), const),
            pl.BlockSpec((1, N_ASSETS), const),
        ],
        out_specs=[
            pl.BlockSpec((tile_rows, N_ASSETS), lambda i: (i, 0)),
            pl.BlockSpec((tile_rows, N_ASSETS), lambda i: (i, 0)),
        ],
        out_shape=[
            jax.ShapeDtypeStruct((batch, N_ASSETS), jnp.float32),
            jax.ShapeDtypeStruct((batch, N_ASSETS), jnp.float32),
        ],
        compiler_params=pltpu.CompilerParams(
            dimension_semantics=("arbitrary",),
        ),
    )(x, W1, b1r, W2, b2r, wh_all, bh_all, lsr)
    return (mean, std)
